# baseline (speedup 1.0000x reference)
import jax
import jax.numpy as jnp
from jax import lax
from jax.experimental import pallas as pl
from jax.experimental.pallas import tpu as pltpu
from jax.experimental.pallas import tpu_sc as plsc

_TOK_ID = 12345
_STRENGTH = 18.0

_SEQ_BLK = 2048


def _sc_rows_body(x_ref, li_ref, w_ref, rows_ref, li_v, idx_v, rowbuf, wv,
                  accbuf, sem):
    nc = 2
    c = lax.axis_index("c")
    s = lax.axis_index("s")
    wid = s * nc + c

    pltpu.sync_copy(li_ref, li_v)
    pltpu.sync_copy(w_ref.at[pl.ds(_TOK_ID, 1), :], wv)

    iota = lax.iota(jnp.int32, 16)
    li_own = plsc.load_gather(li_v, [jnp.full((16,), wid, jnp.int32)])
    idx_v[...] = wid * 2048 + li_own

    pltpu.async_copy(x_ref.at[idx_v], rowbuf, sem).wait()

    def _sumsq(j, acc):
        wc = wv[0, pl.ds(pl.multiple_of(j * 16, 16), 16)]
        return acc + wc * wc

    acc = lax.fori_loop(0, 64, _sumsq, jnp.zeros((16,), jnp.float32))
    for k in (1, 2, 4, 8):
        accbuf[...] = acc
        acc = acc + plsc.load_gather(accbuf, [lax.bitwise_xor(iota, k)])
    tv = acc

    seed = jnp.full((16,), 0x5F3759DF, jnp.int32) - (plsc.bitcast(tv, jnp.int32) >> 1)
    y = plsc.bitcast(seed, jnp.float32)
    for _ in range(4):
        y = y * (1.5 - 0.5 * tv * y * y)
    sv = _STRENGTH * y

    def _graft(j, carry):
        sl = pl.ds(pl.multiple_of(j * 16, 16), 16)
        rowbuf[0, sl] = rowbuf[0, sl] + wv[0, sl] * sv
        return carry

    lax.fori_loop(0, 64, _graft, 0)

    pltpu.sync_copy(rowbuf.at[pl.ds(0, 1), :], rows_ref.at[pl.ds(wid, 1), :])


def _sc_rows(xf, last_indices, lm_head_weight):
    D = xf.shape[1]
    mesh = plsc.VectorSubcoreMesh(core_axis_name="c", subcore_axis_name="s")
    return pl.kernel(
        _sc_rows_body,
        out_type=jax.ShapeDtypeStruct((32, D), jnp.float32),
        mesh=mesh,
        scratch_types=[
            pltpu.VMEM((32,), jnp.int32),
            pltpu.VMEM((16,), jnp.int32),
            pltpu.VMEM((16, D), jnp.float32),
            pltpu.VMEM((1, D), jnp.float32),
            pltpu.VMEM((16,), jnp.float32),
            pltpu.SemaphoreType.DMA,
        ],
        compiler_params=pltpu.CompilerParams(needs_layout_passes=False),
    )(xf, last_indices, lm_head_weight)


def _copy_graft_body(li_ref, x_ref, rows_ref, o_ref):
    b = pl.program_id(0)
    o_ref[...] = x_ref[...]
    li = li_ref[b]
    o_ref[pl.ds(li, 1), :] = rows_ref[pl.ds(b, 1), :]


def _tc_copy_graft(x, last_indices, rows):
    B, S, D = x.shape
    return pl.pallas_call(
        _copy_graft_body,
        grid=(B,),
        in_specs=[
            pl.BlockSpec(memory_space=pltpu.SMEM),
            pl.BlockSpec((None, _SEQ_BLK, D), lambda b: (b, 0, 0)),
            pl.BlockSpec((32, D), lambda b: (0, 0)),
        ],
        out_specs=pl.BlockSpec((None, _SEQ_BLK, D), lambda b: (b, 0, 0)),
        out_shape=jax.ShapeDtypeStruct((B, S, D), x.dtype),
        compiler_params=pltpu.CompilerParams(
            dimension_semantics=("parallel",),
        ),
    )(last_indices, x, rows)


def kernel(x, token_ids, last_indices, lm_head_weight):
    del token_ids
    B, S, D = x.shape
    xf = x.reshape(B * S, D)
    rows = _sc_rows(xf, last_indices, lm_head_weight)
    return _tc_copy_graft(x, last_indices, rows)


# speedup vs baseline: 1.0235x; 1.0235x over previous
import jax
import jax.numpy as jnp
from jax import lax
from jax.experimental import pallas as pl
from jax.experimental.pallas import tpu as pltpu
from jax.experimental.pallas import tpu_sc as plsc

_TOK_ID = 12345
_STRENGTH = 18.0

_SEQ_BLK = 2048


def _sc_rows_body(x_ref, li_ref, w_ref, rows_ref, li_v, idx_v, rowbuf, wv,
                  accbuf, sem):
    nc = 2
    c = lax.axis_index("c")
    s = lax.axis_index("s")
    wid = s * nc + c

    pltpu.sync_copy(li_ref, li_v)
    pltpu.sync_copy(w_ref.at[pl.ds(_TOK_ID, 1), :], wv)

    iota = lax.iota(jnp.int32, 16)
    li_own = plsc.load_gather(li_v, [jnp.full((16,), wid, jnp.int32)])
    idx_v[...] = wid * 2048 + li_own

    pltpu.async_copy(x_ref.at[idx_v], rowbuf, sem).wait()

    def _sumsq(j, acc):
        wc = wv[0, pl.ds(pl.multiple_of(j * 16, 16), 16)]
        return acc + wc * wc

    acc = lax.fori_loop(0, 64, _sumsq, jnp.zeros((16,), jnp.float32))
    for k in (1, 2, 4, 8):
        accbuf[...] = acc
        acc = acc + plsc.load_gather(accbuf, [lax.bitwise_xor(iota, k)])
    tv = acc

    seed = jnp.full((16,), 0x5F3759DF, jnp.int32) - (plsc.bitcast(tv, jnp.int32) >> 1)
    y = plsc.bitcast(seed, jnp.float32)
    for _ in range(4):
        y = y * (1.5 - 0.5 * tv * y * y)
    sv = _STRENGTH * y

    def _graft(j, carry):
        sl = pl.ds(pl.multiple_of(j * 16, 16), 16)
        rowbuf[0, sl] = rowbuf[0, sl] + wv[0, sl] * sv
        return carry

    lax.fori_loop(0, 64, _graft, 0)

    pltpu.sync_copy(rowbuf.at[pl.ds(0, 1), :], rows_ref.at[pl.ds(wid, 1), :])


def _sc_rows(xf, last_indices, lm_head_weight):
    D = xf.shape[1]
    mesh = plsc.VectorSubcoreMesh(core_axis_name="c", subcore_axis_name="s")
    return pl.kernel(
        _sc_rows_body,
        out_type=jax.ShapeDtypeStruct((32, D), jnp.float32),
        mesh=mesh,
        scratch_types=[
            pltpu.VMEM((32,), jnp.int32),
            pltpu.VMEM((16,), jnp.int32),
            pltpu.VMEM((16, D), jnp.float32),
            pltpu.VMEM((1, D), jnp.float32),
            pltpu.VMEM((16,), jnp.float32),
            pltpu.SemaphoreType.DMA,
        ],
        compiler_params=pltpu.CompilerParams(needs_layout_passes=False),
    )(xf, last_indices, lm_head_weight)


def _copy_body(x_ref, o_ref):
    o_ref[...] = x_ref[...]


def _tc_copy(xf):
    V, D = xf.shape
    return pl.pallas_call(
        _copy_body,
        grid=(V // _SEQ_BLK,),
        in_specs=[pl.BlockSpec((_SEQ_BLK, D), lambda i: (i, 0))],
        out_specs=pl.BlockSpec((_SEQ_BLK, D), lambda i: (i, 0)),
        out_shape=jax.ShapeDtypeStruct((V, D), xf.dtype),
        compiler_params=pltpu.CompilerParams(
            dimension_semantics=("parallel",),
        ),
    )(xf)


def _merge_body(li_ref, y_hbm, rows_ref, o_hbm, sem):
    del y_hbm
    cps = []
    for b in range(rows_ref.shape[0]):
        r = b * _SEQ_BLK + li_ref[b]
        cp = pltpu.make_async_copy(
            rows_ref.at[pl.ds(b, 1), :], o_hbm.at[pl.ds(r, 1), :], sem)
        cp.start()
        cps.append(cp)
    for cp in cps:
        cp.wait()


def _tc_merge(last_indices, yf, rows):
    V, D = yf.shape
    return pl.pallas_call(
        _merge_body,
        in_specs=[
            pl.BlockSpec(memory_space=pltpu.SMEM),
            pl.BlockSpec(memory_space=pltpu.MemorySpace.HBM),
            pl.BlockSpec(memory_space=pltpu.VMEM),
        ],
        out_specs=pl.BlockSpec(memory_space=pltpu.MemorySpace.HBM),
        out_shape=jax.ShapeDtypeStruct((V, D), yf.dtype),
        input_output_aliases={1: 0},
        scratch_shapes=[pltpu.SemaphoreType.DMA],
    )(last_indices, yf, rows)


def kernel(x, token_ids, last_indices, lm_head_weight):
    del token_ids
    B, S, D = x.shape
    xf = x.reshape(B * S, D)
    rows = _sc_rows(xf, last_indices, lm_head_weight)
    yf = _tc_copy(xf)
    return _tc_merge(last_indices, yf, rows).reshape(B, S, D)
